# Initial kernel scaffold; baseline (speedup 1.0000x reference)
import jax
import jax.numpy as jnp
from jax.experimental import pallas as pl


def kernel(x, adj, W_ih, W_hh, b_ih, b_hh, Wg, bg, Wm, Wf, bf):
    raise NotImplementedError("write your pallas kernel here")


# speedup vs baseline: 8.2222x; 8.2222x over previous
import jax
import jax.numpy as jnp
from jax.experimental import pallas as pl

B = 32
N = 32
HS = 501
NZ = 56
HSP = 512


def _dvae_body(p_ref, x_ref, wih_ref, bih_ref, whh_ref, bhh_ref, wc_ref,
               gidb_ref, wft_ref, bf_ref, out_ref):
    gs = [None] * N
    h_last = None
    for v in range(N):
        if v == 0:
            hsum = jnp.zeros((B, HSP), jnp.float32)
            gh = jnp.broadcast_to(bhh_ref[...], (B, 3 * HSP))
        else:
            pv = p_ref[v]
            acc = pv[:, 0:1] * gs[0]
            for u in range(1, v):
                acc = acc + pv[:, u:u + 1] * gs[u]
            hsum = acc
            gh = jnp.dot(hsum, whh_ref[...],
                         preferred_element_type=jnp.float32) + bhh_ref[...]
        xv = x_ref[v]
        gi = xv * wih_ref[...] + bih_ref[...]
        r = jax.nn.sigmoid(gi[:, :HSP] + gh[:, :HSP])
        z = jax.nn.sigmoid(gi[:, HSP:2 * HSP] + gh[:, HSP:2 * HSP])
        n = jnp.tanh(gi[:, 2 * HSP:] + r * gh[:, 2 * HSP:])
        hv = n + z * (hsum - n)
        if v == N - 1:
            h_last = hv
        else:
            am = jnp.dot(hv, wc_ref[...],
                         preferred_element_type=jnp.float32) + gidb_ref[v:v + 1, :]
            gs[v] = jax.nn.sigmoid(am[:, :HSP]) * am[:, HSP:]
    mu = jnp.dot(h_last, wft_ref[...],
                 preferred_element_type=jnp.float32) + bf_ref[...]
    out_ref[...] = mu


def kernel(x, adj, W_ih, W_hh, b_ih, b_hh, Wg, bg, Wm, Wf, bf):
    f32 = jnp.float32
    allowed = jnp.triu(jnp.ones((N, N), f32), k=1)
    adj_eff = adj.astype(f32) * allowed[None]
    p = jnp.transpose(adj_eff, (2, 0, 1))
    xv3 = jnp.transpose(x, (1, 0))[:, :, None]

    def pad2(w, r, c):
        return jnp.pad(w, ((0, r - w.shape[0]), (0, c - w.shape[1])))

    def pad1(w, c):
        return jnp.pad(w, (0, c - w.shape[0]))

    whh = jnp.concatenate(
        [pad2(W_hh[g * HS:(g + 1) * HS, :].T, HSP, HSP) for g in range(3)],
        axis=1)
    wih = jnp.concatenate(
        [pad1(W_ih[g * HS:(g + 1) * HS, 0], HSP) for g in range(3)])[None, :]
    bih = jnp.concatenate(
        [pad1(b_ih[g * HS:(g + 1) * HS], HSP) for g in range(3)])[None, :]
    bhh = jnp.concatenate(
        [pad1(b_hh[g * HS:(g + 1) * HS], HSP) for g in range(3)])[None, :]
    wc = jnp.concatenate(
        [pad2(Wg[:, :HS].T, HSP, HSP), pad2(Wm[:, :HS].T, HSP, HSP)], axis=1)
    gidb = jnp.concatenate([
        jnp.pad(Wg[:, HS:].T + bg[None, :], ((0, 0), (0, HSP - HS))),
        jnp.pad(Wm[:, HS:].T, ((0, 0), (0, HSP - HS))),
    ], axis=1)
    wft = jnp.pad(Wf.T, ((0, HSP - HS), (0, 0)))
    bfr = bf[None, :]

    mu = pl.pallas_call(
        _dvae_body,
        out_shape=jax.ShapeDtypeStruct((B, NZ), f32),
    )(p, xv3, wih, bih, whh, bhh, wc, gidb, wft, bfr)
    return mu[:, :, None]
